# baseline (speedup 1.0000x reference)
import functools

import jax
import jax.numpy as jnp
from jax import lax
from jax.experimental import pallas as pl
from jax.experimental.pallas import tpu as pltpu
from jax.experimental.pallas import tpu_sc as plsc

_BATCH = 16384
_D = 64
_NC = 2
_NS = 16
_NW = _NC * _NS
_ROWS = _BATCH // _NW
_CH = 128
_NCH = _ROWS // _CH
_LANES = 16


@functools.partial(
    pl.kernel,
    mesh=plsc.VectorSubcoreMesh(core_axis_name="c", subcore_axis_name="s"),
    compiler_params=pltpu.CompilerParams(use_tc_tiling_on_sc=False),
    out_type=[
        jax.ShapeDtypeStruct((_BATCH,), jnp.float32),
        jax.ShapeDtypeStruct((_BATCH,), jnp.float32),
    ],
    scratch_types=[
        pltpu.VMEM((_NCH, _CH), jnp.int32),
        pltpu.VMEM((_NCH, _CH), jnp.int32),
        pltpu.VMEM((_NCH, _CH), jnp.int32),
        pltpu.VMEM((_ROWS, _D), jnp.float32),
        pltpu.VMEM((_ROWS, _D), jnp.float32),
        pltpu.VMEM((_ROWS, _D), jnp.float32),
        pltpu.VMEM((_ROWS,), jnp.float32),
        pltpu.VMEM((_ROWS,), jnp.float32),
        pltpu.SemaphoreType.DMA,
    ],
)
def _mfbpr_sc(user_hbm, item_i_hbm, item_j_hbm, eu_hbm, ei_hbm,
              out_i_hbm, out_j_hbm,
              idx_u, idx_i, idx_j, rows_u, rows_i, rows_j,
              res_i, res_j, sem):
    wid = lax.axis_index("s") * _NC + lax.axis_index("c")

    pltpu.sync_copy(user_hbm.at[wid], idx_u)
    pltpu.sync_copy(item_i_hbm.at[wid], idx_i)
    pltpu.sync_copy(item_j_hbm.at[wid], idx_j)

    copies = []
    for c in range(_NCH):
        copies.append(pltpu.async_copy(
            eu_hbm.at[idx_u.at[c]], rows_u.at[pl.ds(c * _CH, _CH)], sem))
        copies.append(pltpu.async_copy(
            ei_hbm.at[idx_i.at[c]], rows_i.at[pl.ds(c * _CH, _CH)], sem))
        copies.append(pltpu.async_copy(
            ei_hbm.at[idx_j.at[c]], rows_j.at[pl.ds(c * _CH, _CH)], sem))
    for cp in copies:
        cp.wait()

    lane = lax.iota(jnp.int32, _LANES)
    perms = [lane ^ s for s in (8, 4, 2, 1)]

    dnums = lax.GatherDimensionNumbers(
        offset_dims=(), collapsed_slice_dims=(0,), start_index_map=(0,))

    def lperm(v, p):
        return lax.gather(v, p[:, None], dnums, slice_sizes=(1,),
                          mode=lax.GatherScatterMode.PROMISE_IN_BOUNDS)

    def hsum(v):
        for p in perms:
            v = v + lperm(v, p)
        return v

    def body(g, carry):
        vec_i = jnp.zeros((_LANES,), jnp.float32)
        vec_j = jnp.zeros((_LANES,), jnp.float32)
        for k in range(_LANES):
            r = g * _LANES + k
            pi = jnp.zeros((_LANES,), jnp.float32)
            pj = jnp.zeros((_LANES,), jnp.float32)
            for q in range(_D // _LANES):
                u = rows_u[r, pl.ds(q * _LANES, _LANES)]
                vi = rows_i[r, pl.ds(q * _LANES, _LANES)]
                vj = rows_j[r, pl.ds(q * _LANES, _LANES)]
                pi = pi + u * vi
                pj = pj + u * vj
            sel = lane == k
            vec_i = jnp.where(sel, hsum(pi), vec_i)
            vec_j = jnp.where(sel, hsum(pj), vec_j)
        res_i[pl.ds(g * _LANES, _LANES)] = vec_i
        res_j[pl.ds(g * _LANES, _LANES)] = vec_j
        return carry

    lax.fori_loop(0, _ROWS // _LANES, body, 0)

    base = wid * _ROWS
    pltpu.sync_copy(res_i, out_i_hbm.at[pl.ds(base, _ROWS)])
    pltpu.sync_copy(res_j, out_j_hbm.at[pl.ds(base, _ROWS)])


def kernel(user, item_i, item_j, embed_user, embed_item):
    u3 = user.astype(jnp.int32).reshape(_NW, _NCH, _CH)
    i3 = item_i.astype(jnp.int32).reshape(_NW, _NCH, _CH)
    j3 = item_j.astype(jnp.int32).reshape(_NW, _NCH, _CH)
    out_i, out_j = _mfbpr_sc(u3, i3, j3, embed_user, embed_item)
    return (out_i, out_j)


# speedup vs baseline: 2.0623x; 2.0623x over previous
import functools

import jax
import jax.numpy as jnp
from jax import lax
from jax.experimental import pallas as pl
from jax.experimental.pallas import tpu as pltpu
from jax.experimental.pallas import tpu_sc as plsc

_BATCH = 16384
_D = 64
_ROWS_TBL = 1000000
_G = 8
_NG = _ROWS_TBL // _G
_NC = 2
_NS = 16
_NW = _NC * _NS
_ROWS = _BATCH // _NW
_LANES = 16
_C = 16
_NCHK = _ROWS // _C


@functools.partial(
    pl.kernel,
    mesh=plsc.VectorSubcoreMesh(core_axis_name="c", subcore_axis_name="s"),
    out_type=[
        jax.ShapeDtypeStruct((_BATCH,), jnp.float32),
        jax.ShapeDtypeStruct((_BATCH,), jnp.float32),
    ],
    scratch_types=[
        pltpu.VMEM((_ROWS,), jnp.int32),
        pltpu.VMEM((_ROWS,), jnp.int32),
        pltpu.VMEM((_ROWS,), jnp.int32),
        pltpu.VMEM((_C, _G, _D), jnp.float32),
        pltpu.VMEM((_C, _G, _D), jnp.float32),
        pltpu.VMEM((_C, _G, _D), jnp.float32),
        pltpu.VMEM((_ROWS,), jnp.float32),
        pltpu.VMEM((_ROWS,), jnp.float32),
        pltpu.SemaphoreType.DMA,
    ],
)
def _mfbpr_sc(user_hbm, item_i_hbm, item_j_hbm, eu_hbm, ei_hbm,
              out_i_hbm, out_j_hbm,
              raw_u, raw_i, raw_j, buf_u, buf_i, buf_j,
              res_i, res_j, sem):
    wid = lax.axis_index("s") * _NC + lax.axis_index("c")
    base = wid * _ROWS

    pltpu.sync_copy(user_hbm.at[pl.ds(base, _ROWS)], raw_u)
    pltpu.sync_copy(item_i_hbm.at[pl.ds(base, _ROWS)], raw_i)
    pltpu.sync_copy(item_j_hbm.at[pl.ds(base, _ROWS)], raw_j)

    lane = lax.iota(jnp.int32, _LANES)
    perms = [lane ^ p for p in (8, 4, 2, 1)]
    dnums = lax.GatherDimensionNumbers(
        offset_dims=(), collapsed_slice_dims=(0,), start_index_map=(0,))

    def hsum(v):
        for p in perms:
            v = v + lax.gather(v, p[:, None], dnums, slice_sizes=(1,),
                               mode=lax.GatherScatterMode.PROMISE_IN_BOUNDS)
        return v

    def extract(v, k):
        return jnp.squeeze(lax.slice(v, (k,), (k + 1,)))

    def chunk_body(c, carry):
        cb = c * _C
        gv_u = lax.shift_right_logical(raw_u[pl.ds(cb, _LANES)], 3)
        gv_i = lax.shift_right_logical(raw_i[pl.ds(cb, _LANES)], 3)
        gv_j = lax.shift_right_logical(raw_j[pl.ds(cb, _LANES)], 3)
        cps = []
        for k in range(_C):
            cps.append(pltpu.async_copy(
                eu_hbm.at[extract(gv_u, k)], buf_u.at[k], sem))
            cps.append(pltpu.async_copy(
                ei_hbm.at[extract(gv_i, k)], buf_i.at[k], sem))
            cps.append(pltpu.async_copy(
                ei_hbm.at[extract(gv_j, k)], buf_j.at[k], sem))
        for cp in cps:
            cp.wait()

        rv_u = lax.bitwise_and(raw_u[pl.ds(cb, _LANES)], 7)
        rv_i = lax.bitwise_and(raw_i[pl.ds(cb, _LANES)], 7)
        rv_j = lax.bitwise_and(raw_j[pl.ds(cb, _LANES)], 7)
        vec_i = jnp.zeros((_LANES,), jnp.float32)
        vec_j = jnp.zeros((_LANES,), jnp.float32)
        for k in range(_C):
            ru = extract(rv_u, k)
            ri = extract(rv_i, k)
            rj = extract(rv_j, k)
            pi = jnp.zeros((_LANES,), jnp.float32)
            pj = jnp.zeros((_LANES,), jnp.float32)
            for q in range(_D // _LANES):
                off = pl.ds(q * _LANES, _LANES)
                u = buf_u[k, ru, off]
                vi = buf_i[k, ri, off]
                vj = buf_j[k, rj, off]
                pi = pi + u * vi
                pj = pj + u * vj
            sel = lane == k
            vec_i = jnp.where(sel, hsum(pi), vec_i)
            vec_j = jnp.where(sel, hsum(pj), vec_j)
        res_i[pl.ds(cb, _LANES)] = vec_i
        res_j[pl.ds(cb, _LANES)] = vec_j
        return carry

    lax.fori_loop(0, _NCHK, chunk_body, 0)

    pltpu.sync_copy(res_i, out_i_hbm.at[pl.ds(base, _ROWS)])
    pltpu.sync_copy(res_j, out_j_hbm.at[pl.ds(base, _ROWS)])


def kernel(user, item_i, item_j, embed_user, embed_item):
    eu3 = embed_user.reshape(_NG, _G, _D)
    ei3 = embed_item.reshape(_NG, _G, _D)
    out_i, out_j = _mfbpr_sc(user.astype(jnp.int32), item_i.astype(jnp.int32),
                             item_j.astype(jnp.int32), eu3, ei3)
    return (out_i, out_j)
